# Initial kernel scaffold; baseline (speedup 1.0000x reference)
import jax
import jax.numpy as jnp
from jax.experimental import pallas as pl


def kernel(x, edge_attr, edge_index, ef_W1, ef_b1, ef_W2, ef_b2, aw1_W, aw1_b, aw2_W, aw2_b, aw3_W, aw3_b, centers, gamma):
    raise NotImplementedError("write your pallas kernel here")


# speedup vs baseline: 1.3177x; 1.3177x over previous
import functools

import jax
import jax.numpy as jnp
import numpy as np
from jax import lax
from jax.experimental import pallas as pl
from jax.experimental.pallas import tpu as pltpu
from jax.experimental.pallas import tpu_sc as plsc

NC = 2
NS = 16
LANES = 16
CHUNK = 128
ZROWS = 400
OTILES = 5


def _interleave_perm(dim):
    p = np.empty((dim,), np.int32)
    for j in range(dim // 32):
        for k in range(16):
            p[32 * j + k] = 32 * j + 2 * k
            p[32 * j + 16 + k] = 32 * j + 2 * k + 1
    return p


def _dense_bias_kernel(x_ref, w_ref, b_ref, o_ref):
    o_ref[...] = (
        jnp.dot(x_ref[...], w_ref[...], preferred_element_type=jnp.float32)
        + b_ref[...]
    )


def _edge_feat_kernel(attr_ref, m_ref, c_ref, g_ref, w1_ref, b1_ref, w2_ref,
                      b2_ref, o_ref):
    t = attr_ref[...]
    g = g_ref[0, 0]
    d = c_ref[...] - t
    a = jnp.exp(-g * d * d)
    z = jnp.dot(a, w1_ref[...], preferred_element_type=jnp.float32) + b1_ref[...]
    z = z * jax.nn.sigmoid(z)
    z = jnp.dot(z, w2_ref[...], preferred_element_type=jnp.float32) + b2_ref[...]
    o_ref[...] = z * jax.nn.sigmoid(z) * m_ref[...]


def _final_kernel(x_ref, agg_ref, w2_ref, b2_ref, w3_ref, b3_ref, o_ref):
    agg = (agg_ref[0].astype(jnp.float32) + agg_ref[1].astype(jnp.float32))
    z = jnp.dot(agg, w2_ref[...], preferred_element_type=jnp.float32) + b2_ref[...]
    z = z * jax.nn.sigmoid(z)
    o_ref[...] = (
        x_ref[...]
        + jnp.dot(z, w3_ref[...], preferred_element_type=jnp.float32)
        + b3_ref[...]
    )


def _make_sc_scatter(n_nodes, dim, cpt):
    mesh = plsc.VectorSubcoreMesh(core_axis_name="c", subcore_axis_name="s")
    zsegs = n_nodes // (OTILES * ZROWS)
    orows = n_nodes // OTILES

    @functools.partial(
        pl.kernel,
        out_type=jax.ShapeDtypeStruct((NC, n_nodes, dim), jnp.bfloat16),
        mesh=mesh,
        scratch_types=[
            pltpu.VMEM((cpt, CHUNK), jnp.int32),
            pltpu.VMEM((cpt, CHUNK), jnp.int32),
            pltpu.VMEM((CHUNK, dim), jnp.float32),
            pltpu.VMEM((CHUNK, dim), jnp.float32),
            pltpu.VMEM((CHUNK, dim), jnp.bfloat16),
            pltpu.VMEM_SHARED((n_nodes, dim), jnp.bfloat16),
            pltpu.SemaphoreType.DMA,
        ],
        compiler_params=pltpu.CompilerParams(needs_layout_passes=False,
                                             use_tc_tiling_on_sc=False),
    )
    def sc_kernel(h_hbm, a_hbm, src_hbm, dst_hbm, z_hbm, out_hbm,
                  src_v, dst_v, a_v, hj_v, mb_v, agg_sh, sem):
        cid = lax.axis_index("c")
        sid = lax.axis_index("s")
        tile = cid * NS + sid

        @pl.when(sid < OTILES)
        def _():
            def zb(j, c):
                pltpu.sync_copy(
                    z_hbm, agg_sh.at[pl.ds(sid * orows + j * ZROWS, ZROWS)])
                return c
            lax.fori_loop(0, zsegs, zb, 0)

        pltpu.sync_copy(src_hbm.at[pl.ds(tile * cpt, cpt)], src_v)
        pltpu.sync_copy(dst_hbm.at[pl.ds(tile * cpt, cpt)], dst_v)
        plsc.subcore_barrier()

        def chunk_body(k, carry):
            pltpu.async_copy(h_hbm.at[src_v.at[k]], hj_v, sem).wait()
            pltpu.sync_copy(
                a_hbm.at[pl.ds((tile * cpt + k) * CHUNK, CHUNK)], a_v)

            def row_body(i, c2):
                for j in range(dim // 32):
                    s0 = pl.ds(32 * j, LANES)
                    s1 = pl.ds(32 * j + LANES, LANES)
                    m0 = hj_v[i, s0] * a_v[i, s0]
                    m1 = hj_v[i, s1] * a_v[i, s1]
                    mb_v[i, pl.ds(32 * j, 32)] = plsc.pack(
                        m0, m1, format=plsc.PackFormat.INTERLEAVED)
                return c2

            lax.fori_loop(0, CHUNK, row_body, 0)
            pltpu.sync_copy(mb_v, agg_sh.at[dst_v.at[k]], add=True)
            return carry

        lax.fori_loop(0, cpt, chunk_body, 0)
        plsc.subcore_barrier()

        @pl.when(sid < OTILES)
        def _():
            pltpu.sync_copy(
                agg_sh.at[pl.ds(sid * orows, orows)],
                out_hbm.at[cid, pl.ds(sid * orows, orows)],
            )

    return sc_kernel


def kernel(x, edge_attr, edge_index, ef_W1, ef_b1, ef_W2, ef_b2,
           aw1_W, aw1_b, aw2_W, aw2_b, aw3_W, aw3_b, centers, gamma):
    n, dim = x.shape
    e = edge_index.shape[1]
    n_rbf = centers.shape[0]

    nw = NC * NS
    cpt = -(-e // (nw * CHUNK))
    cpt = -(-cpt // 8) * 8
    e_pad = nw * CHUNK * cpt
    pad = e_pad - e

    perm = _interleave_perm(dim)

    ei = edge_index.astype(jnp.int32)
    src2 = jnp.concatenate([ei[0], jnp.zeros((pad,), jnp.int32)]).reshape(
        e_pad // CHUNK, CHUNK)
    dst2 = jnp.concatenate([ei[1], jnp.zeros((pad,), jnp.int32)]).reshape(
        e_pad // CHUNK, CHUNK)
    attr_pad = jnp.concatenate(
        [edge_attr.astype(jnp.float32), jnp.zeros((pad, 1), jnp.float32)])
    mask = jnp.concatenate(
        [jnp.ones((e, 1), jnp.float32), jnp.zeros((pad, 1), jnp.float32)])
    zeros_blk = jnp.zeros((ZROWS, dim), jnp.bfloat16)

    rb = 1000
    h = pl.pallas_call(
        _dense_bias_kernel,
        grid=(n // rb,),
        in_specs=[
            pl.BlockSpec((rb, dim), lambda i: (i, 0)),
            pl.BlockSpec((dim, dim), lambda i: (0, 0)),
            pl.BlockSpec((1, dim), lambda i: (0, 0)),
        ],
        out_specs=pl.BlockSpec((rb, dim), lambda i: (i, 0)),
        out_shape=jax.ShapeDtypeStruct((n, dim), jnp.float32),
    )(x, aw1_W.T[:, perm], aw1_b[perm].reshape(1, dim))

    blk = 2048
    assert e_pad % blk == 0
    a_feat = pl.pallas_call(
        _edge_feat_kernel,
        grid=(e_pad // blk,),
        in_specs=[
            pl.BlockSpec((blk, 1), lambda i: (i, 0)),
            pl.BlockSpec((blk, 1), lambda i: (i, 0)),
            pl.BlockSpec((1, n_rbf), lambda i: (0, 0)),
            pl.BlockSpec(memory_space=pltpu.SMEM),
            pl.BlockSpec((n_rbf, dim), lambda i: (0, 0)),
            pl.BlockSpec((1, dim), lambda i: (0, 0)),
            pl.BlockSpec((dim, dim), lambda i: (0, 0)),
            pl.BlockSpec((1, dim), lambda i: (0, 0)),
        ],
        out_specs=pl.BlockSpec((blk, dim), lambda i: (i, 0)),
        out_shape=jax.ShapeDtypeStruct((e_pad, dim), jnp.float32),
    )(attr_pad, mask, centers.reshape(1, n_rbf),
      jnp.asarray(gamma, jnp.float32).reshape(1, 1),
      ef_W1.T, ef_b1.reshape(1, dim), ef_W2.T[:, perm],
      ef_b2[perm].reshape(1, dim))

    agg2 = _make_sc_scatter(n, dim, cpt)(h, a_feat, src2, dst2, zeros_blk)

    out = pl.pallas_call(
        _final_kernel,
        grid=(n // rb,),
        in_specs=[
            pl.BlockSpec((rb, dim), lambda i: (i, 0)),
            pl.BlockSpec((NC, rb, dim), lambda i: (0, i, 0)),
            pl.BlockSpec((dim, dim), lambda i: (0, 0)),
            pl.BlockSpec((1, dim), lambda i: (0, 0)),
            pl.BlockSpec((dim, dim), lambda i: (0, 0)),
            pl.BlockSpec((1, dim), lambda i: (0, 0)),
        ],
        out_specs=pl.BlockSpec((rb, dim), lambda i: (i, 0)),
        out_shape=jax.ShapeDtypeStruct((n, dim), jnp.float32),
    )(x, agg2, aw2_W.T, aw2_b.reshape(1, dim), aw3_W.T, aw3_b.reshape(1, dim))

    return out
